# Initial kernel scaffold; baseline (speedup 1.0000x reference)
import jax
import jax.numpy as jnp
from jax.experimental import pallas as pl


def kernel(x, adj, W1, b1, W2, b2):
    raise NotImplementedError("write your pallas kernel here")


# speedup vs baseline: 1.8386x; 1.8386x over previous
import functools

import jax
import jax.numpy as jnp
from jax.experimental import pallas as pl
from jax.experimental.pallas import tpu as pltpu

_N = 10000
_D = 128
_BM = 200
_NI = _N // _BM


def _gcn_body(x_ref, adj_ref, w1_ref, b1_ref, w2_ref, b2_ref, out_ref,
              supp_ref, h_ref, cs_ref):
    i = pl.program_id(0)

    @pl.when(i == 0)
    def _init():
        supp_ref[...] = jnp.dot(x_ref[...], w1_ref[...],
                                preferred_element_type=jnp.float32)
        cs_ref[...] = jnp.zeros_like(cs_ref)

    a = adj_ref[...]
    ones_row = jnp.ones((1, _BM), dtype=jnp.float32)
    cs_ref[...] += jnp.dot(ones_row, a, preferred_element_type=jnp.float32)
    h = jnp.dot(a, supp_ref[...], preferred_element_type=jnp.float32)
    h_ref[pl.ds(i * _BM, _BM), :] = jnp.maximum(h + b1_ref[...], 0.0)

    @pl.when(i == _NI - 1)
    def _epilogue():
        t = jnp.dot(cs_ref[...], h_ref[...],
                    preferred_element_type=jnp.float32)
        u = jnp.dot(t, w2_ref[...], preferred_element_type=jnp.float32)
        out_ref[...] = u * (1.0 / _N) + b2_ref[...]


@functools.partial(jax.jit, static_argnames=())
def kernel(x, adj, W1, b1, W2, b2):
    out = pl.pallas_call(
        _gcn_body,
        grid=(_NI,),
        in_specs=[
            pl.BlockSpec((_N, _D), lambda i: (0, 0)),
            pl.BlockSpec((_BM, _N), lambda i: (i, 0)),
            pl.BlockSpec((_D, _D), lambda i: (0, 0)),
            pl.BlockSpec((1, _D), lambda i: (0, 0)),
            pl.BlockSpec((_D, _D), lambda i: (0, 0)),
            pl.BlockSpec((1, _D), lambda i: (0, 0)),
        ],
        out_specs=pl.BlockSpec((1, _D), lambda i: (0, 0)),
        out_shape=jax.ShapeDtypeStruct((1, _D), jnp.float32),
        scratch_shapes=[
            pltpu.VMEM((_N, _D), jnp.float32),
            pltpu.VMEM((_N, _D), jnp.float32),
            pltpu.VMEM((1, _N), jnp.float32),
        ],
        compiler_params=pltpu.CompilerParams(
            dimension_semantics=("arbitrary",),
        ),
    )(x, adj, W1, b1.reshape(1, _D), W2, b2.reshape(1, _D))
    return out.reshape(_D)


# speedup vs baseline: 2.0437x; 1.1115x over previous
import functools

import jax
import jax.numpy as jnp
from jax.experimental import pallas as pl
from jax.experimental.pallas import tpu as pltpu

_N = 10000
_D = 128
_BM = 400
_NI = _N // _BM


def _gcn_body(x_ref, adj_ref, w1_ref, b1_ref, w2_ref, b2_ref, out_ref,
              supp_ref, h_ref, cs_ref):
    i = pl.program_id(0)

    @pl.when(i == 0)
    def _init():
        supp_ref[...] = jnp.dot(x_ref[...], w1_ref[...],
                                preferred_element_type=jnp.float32)
        cs_ref[...] = jnp.zeros_like(cs_ref)

    a = adj_ref[...]
    ones_row = jnp.ones((1, _BM), dtype=jnp.float32)
    cs_ref[...] += jnp.dot(ones_row, a, preferred_element_type=jnp.float32)
    h = jnp.dot(a, supp_ref[...], preferred_element_type=jnp.float32)
    h_ref[pl.ds(i * _BM, _BM), :] = jnp.maximum(h + b1_ref[...], 0.0)

    @pl.when(i == _NI - 1)
    def _epilogue():
        t = jnp.dot(cs_ref[...], h_ref[...],
                    preferred_element_type=jnp.float32)
        u = jnp.dot(t, w2_ref[...], preferred_element_type=jnp.float32)
        out_ref[...] = u * (1.0 / _N) + b2_ref[...]


@functools.partial(jax.jit, static_argnames=())
def kernel(x, adj, W1, b1, W2, b2):
    out = pl.pallas_call(
        _gcn_body,
        grid=(_NI,),
        in_specs=[
            pl.BlockSpec((_N, _D), lambda i: (0, 0)),
            pl.BlockSpec((_BM, _N), lambda i: (i, 0)),
            pl.BlockSpec((_D, _D), lambda i: (0, 0)),
            pl.BlockSpec((1, _D), lambda i: (0, 0)),
            pl.BlockSpec((_D, _D), lambda i: (0, 0)),
            pl.BlockSpec((1, _D), lambda i: (0, 0)),
        ],
        out_specs=pl.BlockSpec((1, _D), lambda i: (0, 0)),
        out_shape=jax.ShapeDtypeStruct((1, _D), jnp.float32),
        scratch_shapes=[
            pltpu.VMEM((_N, _D), jnp.float32),
            pltpu.VMEM((_N, _D), jnp.float32),
            pltpu.VMEM((1, _N), jnp.float32),
        ],
        compiler_params=pltpu.CompilerParams(
            dimension_semantics=("arbitrary",),
        ),
    )(x, adj, W1, b1.reshape(1, _D), W2, b2.reshape(1, _D))
    return out.reshape(_D)
